# speedup vs baseline: 4.7231x; 4.7231x over previous
import functools
import math

import jax
import jax.numpy as jnp
from jax import lax
from jax.experimental import pallas as pl
from jax.experimental.pallas import tpu as pltpu
from jax.experimental.pallas import tpu_sc as plsc

D_MODEL = 128
_SCALE = math.sqrt(float(D_MODEL))
_NC = 2
_NS = 16
_NW = _NC * _NS
_CHUNK = 128
_LANES = 16


def _embed(xf, table):
    B = xf.shape[0]
    b_per_w = B // _NW
    n_chunks = b_per_w // _CHUNK
    mesh = plsc.VectorSubcoreMesh(core_axis_name="c", subcore_axis_name="s")

    @functools.partial(
        pl.kernel,
        out_type=jax.ShapeDtypeStruct((B, D_MODEL), jnp.float32),
        mesh=mesh,
        scratch_types=[
            pltpu.VMEM((b_per_w,), jnp.int32),
            pltpu.VMEM((_CHUNK, D_MODEL), jnp.float32),
            pltpu.SemaphoreType.DMA,
        ],
    )
    def k(idx_hbm, table_hbm, out_hbm, idx_v, rows_v, sem):
        wid = lax.axis_index("s") * _NC + lax.axis_index("c")
        base = wid * b_per_w
        pltpu.sync_copy(idx_hbm.at[pl.ds(base, b_per_w)], idx_v)

        def chunk_body(c, carry):
            off = c * _CHUNK
            pltpu.async_copy(
                table_hbm.at[idx_v.at[pl.ds(off, _CHUNK)]], rows_v, sem
            ).wait()

            def row_body(r, carry2):
                for j in range(D_MODEL // _LANES):
                    sl = pl.ds(j * _LANES, _LANES)
                    rows_v[r, sl] = rows_v[r, sl] * _SCALE
                return carry2

            lax.fori_loop(0, _CHUNK, row_body, 0)
            pltpu.sync_copy(rows_v, out_hbm.at[pl.ds(base + off, _CHUNK)])
            return carry

        lax.fori_loop(0, n_chunks, chunk_body, 0)

    return k(xf, table)


def kernel(x, table):
    b, s = x.shape
    xf = x.reshape(b * s)
    out = _embed(xf, table)
    return out.reshape(b, s, D_MODEL)


# speedup vs baseline: 7.8647x; 1.6651x over previous
import functools
import math

import jax
import jax.numpy as jnp
from jax import lax
from jax.experimental import pallas as pl
from jax.experimental.pallas import tpu as pltpu
from jax.experimental.pallas import tpu_sc as plsc

D_MODEL = 128
_SCALE = math.sqrt(float(D_MODEL))
_NC = 2
_NS = 16
_NW = _NC * _NS
_CHUNK = 128
_LANES = 16
_NBUF = 2


def _embed(xf, table):
    B = xf.shape[0]
    b_per_w = B // _NW
    n_chunks = b_per_w // _CHUNK
    n_groups = n_chunks // _NBUF
    mesh = plsc.VectorSubcoreMesh(core_axis_name="c", subcore_axis_name="s")

    @functools.partial(
        pl.kernel,
        out_type=jax.ShapeDtypeStruct((B, D_MODEL), jnp.float32),
        mesh=mesh,
        scratch_types=[
            pltpu.VMEM((b_per_w,), jnp.int32),
            pltpu.VMEM((_NBUF, _CHUNK, D_MODEL), jnp.float32),
            pltpu.VMEM((_NBUF, _CHUNK, D_MODEL), jnp.float32),
        ]
        + [pltpu.SemaphoreType.DMA] * (2 * _NBUF),
    )
    def k(idx_hbm, table_hbm, out_hbm, idx_v, in_v, out_v, *sems):
        gsem = sems[:_NBUF]
        osem = sems[_NBUF:]
        wid = lax.axis_index("s") * _NC + lax.axis_index("c")
        base = wid * b_per_w
        pltpu.sync_copy(idx_hbm.at[pl.ds(base, b_per_w)], idx_v)

        def gather(off, b):
            pltpu.make_async_copy(
                table_hbm.at[idx_v.at[pl.ds(off, _CHUNK)]], in_v.at[b], gsem[b]
            ).start()

        def gather_wait(b):
            pltpu.make_async_copy(
                table_hbm.at[idx_v.at[pl.ds(0, _CHUNK)]], in_v.at[b], gsem[b]
            ).wait()

        def out_start(off, b):
            pltpu.make_async_copy(
                out_v.at[b], out_hbm.at[pl.ds(base + off, _CHUNK)], osem[b]
            ).start()

        def out_wait(b):
            pltpu.make_async_copy(
                out_v.at[b], out_hbm.at[pl.ds(base, _CHUNK)], osem[b]
            ).wait()

        for b in range(_NBUF):
            gather(b * _CHUNK, b)

        def group_body(g, carry):
            for b in range(_NBUF):
                c = g * _NBUF + b
                off = c * _CHUNK
                gather_wait(b)

                @pl.when(c >= _NBUF)
                def _():
                    out_wait(b)

                def row_body(r, carry2):
                    for j in range(D_MODEL // _LANES):
                        sl = pl.ds(j * _LANES, _LANES)
                        out_v[b, r, sl] = in_v[b, r, sl] * _SCALE
                    return carry2

                lax.fori_loop(0, _CHUNK, row_body, 0)

                @pl.when(c + _NBUF < n_chunks)
                def _():
                    gather(off + _NBUF * _CHUNK, b)

                out_start(off, b)
            return carry

        lax.fori_loop(0, n_groups, group_body, 0)
        for b in range(_NBUF):
            out_wait(b)

    return k(xf, table)


def kernel(x, table):
    b, s = x.shape
    xf = x.reshape(b * s)
    out = _embed(xf, table)
    return out.reshape(b, s, D_MODEL)


# speedup vs baseline: 8.0356x; 1.0217x over previous
import functools
import math

import jax
import jax.numpy as jnp
from jax import lax
from jax.experimental import pallas as pl
from jax.experimental.pallas import tpu as pltpu
from jax.experimental.pallas import tpu_sc as plsc

D_MODEL = 128
_SCALE = math.sqrt(float(D_MODEL))
_NC = 2
_NS = 16
_NW = _NC * _NS
_CHUNK = 64
_LANES = 16
_NBUF = 4


def _embed(xf, table):
    B = xf.shape[0]
    b_per_w = B // _NW
    n_chunks = b_per_w // _CHUNK
    n_groups = n_chunks // _NBUF
    mesh = plsc.VectorSubcoreMesh(core_axis_name="c", subcore_axis_name="s")

    @functools.partial(
        pl.kernel,
        out_type=jax.ShapeDtypeStruct((B, D_MODEL), jnp.float32),
        mesh=mesh,
        scratch_types=[
            pltpu.VMEM((b_per_w,), jnp.int32),
            pltpu.VMEM((_NBUF, _CHUNK, D_MODEL), jnp.float32),
            pltpu.VMEM((_NBUF, _CHUNK, D_MODEL), jnp.float32),
        ]
        + [pltpu.SemaphoreType.DMA] * (2 * _NBUF),
    )
    def k(idx_hbm, table_hbm, out_hbm, idx_v, in_v, out_v, *sems):
        gsem = sems[:_NBUF]
        osem = sems[_NBUF:]
        wid = lax.axis_index("s") * _NC + lax.axis_index("c")
        base = wid * b_per_w
        pltpu.sync_copy(idx_hbm.at[pl.ds(base, b_per_w)], idx_v)

        def gather(off, b):
            pltpu.make_async_copy(
                table_hbm.at[idx_v.at[pl.ds(off, _CHUNK)]], in_v.at[b], gsem[b]
            ).start()

        def gather_wait(b):
            pltpu.make_async_copy(
                table_hbm.at[idx_v.at[pl.ds(0, _CHUNK)]], in_v.at[b], gsem[b]
            ).wait()

        def out_start(off, b):
            pltpu.make_async_copy(
                out_v.at[b], out_hbm.at[pl.ds(base + off, _CHUNK)], osem[b]
            ).start()

        def out_wait(b):
            pltpu.make_async_copy(
                out_v.at[b], out_hbm.at[pl.ds(base, _CHUNK)], osem[b]
            ).wait()

        for b in range(_NBUF):
            gather(b * _CHUNK, b)

        def group_body(g, carry):
            for b in range(_NBUF):
                c = g * _NBUF + b
                off = c * _CHUNK
                gather_wait(b)

                @pl.when(c >= _NBUF)
                def _():
                    out_wait(b)

                def row_body(r, carry2):
                    for j in range(D_MODEL // _LANES):
                        sl = pl.ds(j * _LANES, _LANES)
                        out_v[b, r, sl] = in_v[b, r, sl] * _SCALE
                    return carry2

                lax.fori_loop(0, _CHUNK, row_body, 0)

                @pl.when(c + _NBUF < n_chunks)
                def _():
                    gather(off + _NBUF * _CHUNK, b)

                out_start(off, b)
            return carry

        lax.fori_loop(0, n_groups, group_body, 0)
        for b in range(_NBUF):
            out_wait(b)

    return k(xf, table)


def kernel(x, table):
    b, s = x.shape
    xf = x.reshape(b * s)
    out = _embed(xf, table)
    return out.reshape(b, s, D_MODEL)


# speedup vs baseline: 8.0508x; 1.0019x over previous
import functools
import math

import jax
import jax.numpy as jnp
from jax import lax
from jax.experimental import pallas as pl
from jax.experimental.pallas import tpu as pltpu
from jax.experimental.pallas import tpu_sc as plsc

D_MODEL = 128
_SCALE = math.sqrt(float(D_MODEL))
_NC = 2
_NS = 16
_NW = _NC * _NS
_CHUNK = 64
_LANES = 16
_NBUF = 5


def _embed(xf, table):
    B = xf.shape[0]
    b_per_w = B // _NW
    n_chunks = b_per_w // _CHUNK
    n_groups = n_chunks // _NBUF
    mesh = plsc.VectorSubcoreMesh(core_axis_name="c", subcore_axis_name="s")

    @functools.partial(
        pl.kernel,
        out_type=jax.ShapeDtypeStruct((B, D_MODEL), jnp.float32),
        mesh=mesh,
        scratch_types=[
            pltpu.VMEM((b_per_w,), jnp.int32),
            pltpu.VMEM((_NBUF, _CHUNK, D_MODEL), jnp.float32),
            pltpu.VMEM((_NBUF, _CHUNK, D_MODEL), jnp.float32),
        ]
        + [pltpu.SemaphoreType.DMA] * (2 * _NBUF),
    )
    def k(idx_hbm, table_hbm, out_hbm, idx_v, in_v, out_v, *sems):
        gsem = sems[:_NBUF]
        osem = sems[_NBUF:]
        wid = lax.axis_index("s") * _NC + lax.axis_index("c")
        base = wid * b_per_w
        pltpu.sync_copy(idx_hbm.at[pl.ds(base, b_per_w)], idx_v)

        def gather(off, b):
            pltpu.make_async_copy(
                table_hbm.at[idx_v.at[pl.ds(off, _CHUNK)]], in_v.at[b], gsem[b]
            ).start()

        def gather_wait(b):
            pltpu.make_async_copy(
                table_hbm.at[idx_v.at[pl.ds(0, _CHUNK)]], in_v.at[b], gsem[b]
            ).wait()

        def out_start(off, b):
            pltpu.make_async_copy(
                out_v.at[b], out_hbm.at[pl.ds(base + off, _CHUNK)], osem[b]
            ).start()

        def out_wait(b):
            pltpu.make_async_copy(
                out_v.at[b], out_hbm.at[pl.ds(base, _CHUNK)], osem[b]
            ).wait()

        for b in range(_NBUF):
            gather(b * _CHUNK, b)

        def group_body(g, carry):
            for b in range(_NBUF):
                c = g * _NBUF + b
                off = c * _CHUNK
                gather_wait(b)

                @pl.when(c >= _NBUF)
                def _():
                    out_wait(b)

                def row_body(r, carry2):
                    for j in range(D_MODEL // _LANES):
                        sl = pl.ds(j * _LANES, _LANES)
                        out_v[b, r, sl] = in_v[b, r, sl] * _SCALE
                    return carry2

                lax.fori_loop(0, _CHUNK, row_body, 0)
                out_start(off, b)

                @pl.when(c + _NBUF < n_chunks)
                def _():
                    gather(off + _NBUF * _CHUNK, b)
            return carry

        lax.fori_loop(0, n_groups, group_body, 0)
        for b in range(_NBUF):
            out_wait(b)

    return k(xf, table)


def kernel(x, table):
    b, s = x.shape
    xf = x.reshape(b * s)
    out = _embed(xf, table)
    return out.reshape(b, s, D_MODEL)


# speedup vs baseline: 8.0556x; 1.0006x over previous
import functools
import math

import jax
import jax.numpy as jnp
from jax import lax
from jax.experimental import pallas as pl
from jax.experimental.pallas import tpu as pltpu
from jax.experimental.pallas import tpu_sc as plsc

D_MODEL = 128
_SCALE = math.sqrt(float(D_MODEL))
_NC = 2
_NS = 16
_NW = _NC * _NS
_CHUNK = 32
_LANES = 16
_NBUF = 8


def _embed(xf, table):
    B = xf.shape[0]
    b_per_w = B // _NW
    n_chunks = b_per_w // _CHUNK
    n_groups = n_chunks // _NBUF
    mesh = plsc.VectorSubcoreMesh(core_axis_name="c", subcore_axis_name="s")

    @functools.partial(
        pl.kernel,
        out_type=jax.ShapeDtypeStruct((B, D_MODEL), jnp.float32),
        mesh=mesh,
        scratch_types=[
            pltpu.VMEM((b_per_w,), jnp.int32),
            pltpu.VMEM((_NBUF, _CHUNK, D_MODEL), jnp.float32),
            pltpu.VMEM((_NBUF, _CHUNK, D_MODEL), jnp.float32),
        ]
        + [pltpu.SemaphoreType.DMA] * (2 * _NBUF),
    )
    def k(idx_hbm, table_hbm, out_hbm, idx_v, in_v, out_v, *sems):
        gsem = sems[:_NBUF]
        osem = sems[_NBUF:]
        wid = lax.axis_index("s") * _NC + lax.axis_index("c")
        base = wid * b_per_w
        pltpu.sync_copy(idx_hbm.at[pl.ds(base, b_per_w)], idx_v)

        def gather(off, b):
            pltpu.make_async_copy(
                table_hbm.at[idx_v.at[pl.ds(off, _CHUNK)]], in_v.at[b], gsem[b]
            ).start()

        def gather_wait(b):
            pltpu.make_async_copy(
                table_hbm.at[idx_v.at[pl.ds(0, _CHUNK)]], in_v.at[b], gsem[b]
            ).wait()

        def out_start(off, b):
            pltpu.make_async_copy(
                out_v.at[b], out_hbm.at[pl.ds(base + off, _CHUNK)], osem[b]
            ).start()

        def out_wait(b):
            pltpu.make_async_copy(
                out_v.at[b], out_hbm.at[pl.ds(base, _CHUNK)], osem[b]
            ).wait()

        for b in range(_NBUF):
            gather(b * _CHUNK, b)

        def group_body(g, carry):
            for b in range(_NBUF):
                c = g * _NBUF + b
                off = c * _CHUNK
                gather_wait(b)

                @pl.when(c >= _NBUF)
                def _():
                    out_wait(b)

                def row_body(r, carry2):
                    for j in range(D_MODEL // _LANES):
                        sl = pl.ds(j * _LANES, _LANES)
                        out_v[b, r, sl] = in_v[b, r, sl] * _SCALE
                    return carry2

                lax.fori_loop(0, _CHUNK, row_body, 0)
                out_start(off, b)

                @pl.when(c + _NBUF < n_chunks)
                def _():
                    gather(off + _NBUF * _CHUNK, b)
            return carry

        lax.fori_loop(0, n_groups, group_body, 0)
        for b in range(_NBUF):
            out_wait(b)

    return k(xf, table)


def kernel(x, table):
    b, s = x.shape
    xf = x.reshape(b * s)
    out = _embed(xf, table)
    return out.reshape(b, s, D_MODEL)


# speedup vs baseline: 8.0566x; 1.0001x over previous
import functools
import math

import jax
import jax.numpy as jnp
from jax import lax
from jax.experimental import pallas as pl
from jax.experimental.pallas import tpu as pltpu
from jax.experimental.pallas import tpu_sc as plsc

D_MODEL = 128
_SCALE = math.sqrt(float(D_MODEL))
_NC = 2
_NS = 16
_NW = _NC * _NS
_CHUNK = 32
_LANES = 16
_NBUF = 8


def _embed(xf, table):
    B = xf.shape[0]
    b_per_w = B // _NW
    n_chunks = b_per_w // _CHUNK
    n_groups = n_chunks // _NBUF
    mesh = plsc.VectorSubcoreMesh(core_axis_name="c", subcore_axis_name="s")

    @functools.partial(
        pl.kernel,
        out_type=jax.ShapeDtypeStruct((B, D_MODEL), jnp.float32),
        mesh=mesh,
        scratch_types=[
            pltpu.VMEM((b_per_w,), jnp.int32),
            pltpu.VMEM((_NBUF, _CHUNK, D_MODEL), jnp.float32),
            pltpu.VMEM((_NBUF, _CHUNK, D_MODEL), jnp.float32),
        ]
        + [pltpu.SemaphoreType.DMA] * (2 * _NBUF),
    )
    def k(idx_hbm, table_hbm, out_hbm, idx_v, in_v, out_v, *sems):
        gsem = sems[:_NBUF]
        osem = sems[_NBUF:]
        wid = lax.axis_index("s") * _NC + lax.axis_index("c")
        base = wid * b_per_w
        pltpu.sync_copy(idx_hbm.at[pl.ds(base, b_per_w)], idx_v)

        def gather(off, b):
            pltpu.make_async_copy(
                table_hbm.at[idx_v.at[pl.ds(off, _CHUNK)]], in_v.at[b], gsem[b]
            ).start()

        def gather_wait(b):
            pltpu.make_async_copy(
                table_hbm.at[idx_v.at[pl.ds(0, _CHUNK)]], in_v.at[b], gsem[b]
            ).wait()

        def out_start(off, b):
            pltpu.make_async_copy(
                out_v.at[b], out_hbm.at[pl.ds(base + off, _CHUNK)], osem[b]
            ).start()

        def out_wait(b):
            pltpu.make_async_copy(
                out_v.at[b], out_hbm.at[pl.ds(base, _CHUNK)], osem[b]
            ).wait()

        for b in range(_NBUF):
            gather(b * _CHUNK, b)

        def group_body(g, carry):
            for b in range(_NBUF):
                c = g * _NBUF + b
                off = c * _CHUNK
                gather_wait(b)

                @pl.when(c >= _NBUF)
                def _():
                    out_wait(b)

                def row_body(r, carry2):
                    for j in range(D_MODEL // _LANES):
                        sl = pl.ds(j * _LANES, _LANES)
                        out_v[b, r, sl] = in_v[b, r, sl] * _SCALE
                    return carry2

                lax.fori_loop(0, _CHUNK, row_body, 0)
                out_start(off, b)

                @pl.when(c + _NBUF < n_chunks)
                def _():
                    gather(off + _NBUF * _CHUNK, b)
            return carry

        lax.fori_loop(0, n_groups, group_body, 0)
        for b in range(_NBUF):
            out_wait(b)

    return k(xf, table)


def kernel(x, table):
    b, s = x.shape
    xf = x.reshape(b * s).astype(jnp.int32)
    out = _embed(xf, table)
    return out.reshape(b, s, D_MODEL)
